# baseline (speedup 1.0000x reference)
import functools

import jax
import jax.numpy as jnp
from jax import lax
from jax.experimental import pallas as pl
from jax.experimental.pallas import tpu as pltpu
from jax.experimental.pallas import tpu_sc as plsc

N = 10000
C = 32
F = 128
D = 32
ALPHA = 0.2

NW = 32
B = 320
NP = NW * B
IDX_ROWS = (B * C) // 128
HC = B // 16
OC = B // 8
GPH = 2


def _tc_body(h_ref, wi_ref, wj_ref, aw_ref, ab_ref, g_ref, t_ref, s_ref):
    h = h_ref[...]
    g = lax.dot_general(h, wj_ref[...], (((1,), (1,)), ((), ())),
                        preferred_element_type=jnp.float32)
    g_ref[...] = g
    a_i = aw_ref[0, 0:D]
    a_j = aw_ref[0, D:2 * D]
    t_ref[...] = jnp.sum(g * a_j[None, :], axis=1)
    wh = lax.dot_general(h, wi_ref[...], (((1,), (1,)), ((), ())),
                         preferred_element_type=jnp.float32)
    s_ref[...] = jnp.sum(wh * a_i[None, :], axis=1) + ab_ref[0, 0]


def _tc_call(h_pad, W_i, W_j, att_w, att_b):
    BM = 1024
    return pl.pallas_call(
        _tc_body,
        grid=(NP // BM,),
        in_specs=[
            pl.BlockSpec((BM, F), lambda i: (i, 0)),
            pl.BlockSpec((D, F), lambda i: (0, 0)),
            pl.BlockSpec((D, F), lambda i: (0, 0)),
            pl.BlockSpec((1, 2 * D), lambda i: (0, 0)),
            pl.BlockSpec((1, 1), lambda i: (0, 0)),
        ],
        out_specs=[
            pl.BlockSpec((BM, D), lambda i: (i, 0)),
            pl.BlockSpec((BM,), lambda i: (i,)),
            pl.BlockSpec((BM,), lambda i: (i,)),
        ],
        out_shape=[
            jax.ShapeDtypeStruct((NP, D), jnp.float32),
            jax.ShapeDtypeStruct((NP,), jnp.float32),
            jax.ShapeDtypeStruct((NP,), jnp.float32),
        ],
    )(h_pad, W_i, W_j, att_w, att_b.reshape(1, 1))


@functools.partial(
    pl.kernel,
    mesh=plsc.VectorSubcoreMesh(core_axis_name="c", subcore_axis_name="s"),
    compiler_params=pltpu.CompilerParams(
        needs_layout_passes=False, use_tc_tiling_on_sc=False),
    out_type=jax.ShapeDtypeStruct((N, D), jnp.float32),
    scratch_types=[
        pltpu.VMEM((IDX_ROWS, 128), jnp.int32),
        pltpu.VMEM((HC, 16), jnp.float32),
        pltpu.VMEM((NP,), jnp.float32),
        pltpu.VMEM((GPH, 128, D), jnp.float32),
        pltpu.VMEM((GPH, 128, D), jnp.float32),
        pltpu.VMEM((B, D), jnp.float32),
        pltpu.SemaphoreType.DMA,
        pltpu.SemaphoreType.DMA,
    ],
)
def _sc_kernel(g_hbm, g3_hbm, idx_hbm, t_hbm, s_hbm, out_hbm,
               idx_v, s_v, t_v, rows_a, rows_b, out_v, sem_a, sem_b):
    wid = lax.axis_index("s") * 2 + lax.axis_index("c")
    pltpu.sync_copy(idx_hbm.at[wid], idx_v)
    pltpu.sync_copy(s_hbm.at[wid], s_v)
    pltpu.sync_copy(t_hbm, t_v)

    def fire(oc, rows_v, sem):
        occ = jnp.minimum(oc, OC - 1)
        for j in range(GPH):
            pltpu.async_copy(
                g_hbm.at[idx_v.at[occ * GPH + j]], rows_v.at[j], sem)

    def drain(rows_v, sem):
        for j in range(GPH):
            pltpu.make_async_copy(g3_hbm.at[0], rows_v.at[j], sem).wait()

    def compute(q, half, rows_v):
        sv = s_v[q, :]
        for l in range(8):
            g = q * 16 + half * 8 + l
            ir, ic = q * 4 + half * 2 + l // 4, (l % 4) * 32
            i0 = idx_v[ir, pl.ds(ic, 16)]
            i1 = idx_v[ir, pl.ds(ic + 16, 16)]
            tv0 = plsc.load_gather(t_v, [i0])
            tv1 = plsc.load_gather(t_v, [i1])
            sg = sv[half * 8 + l]
            l0 = sg + tv0
            l1 = sg + tv1
            l0 = jnp.where(l0 >= 0, l0, ALPHA * l0)
            l1 = jnp.where(l1 >= 0, l1, ALPHA * l1)
            e0 = jnp.exp(l0)
            e1 = jnp.exp(l1)
            z = jnp.sum(e0) + jnp.sum(e1)
            rinv = jnp.full((16,), 1.0, jnp.float32) / z
            a0 = [jnp.zeros((16,), jnp.float32) for _ in range(4)]
            a1 = [jnp.zeros((16,), jnp.float32) for _ in range(4)]
            for c in range(C):
                wc = e0[c] if c < 16 else e1[c - 16]
                rr, rc = (l * C + c) // 128, (l * C + c) % 128
                a0[c % 4] = a0[c % 4] + wc * rows_v[rr, rc, pl.ds(0, 16)]
                a1[c % 4] = a1[c % 4] + wc * rows_v[rr, rc, pl.ds(16, 16)]
            acc0 = (a0[0] + a0[1]) + (a0[2] + a0[3])
            acc1 = (a1[0] + a1[1]) + (a1[2] + a1[3])
            out_v[g, pl.ds(0, 16)] = acc0 * rinv
            out_v[g, pl.ds(16, 16)] = acc1 * rinv

    fire(0, rows_a, sem_a)

    def body(q, carry):
        oc = 2 * q
        fire(oc + 1, rows_b, sem_b)
        drain(rows_a, sem_a)
        compute(q, 0, rows_a)
        fire(oc + 2, rows_a, sem_a)
        drain(rows_b, sem_b)
        compute(q, 1, rows_b)
        return carry

    lax.fori_loop(0, OC // 2, body, 0)
    drain(rows_a, sem_a)
    base = wid * B

    @pl.when(wid < NW - 1)
    def _():
        pltpu.sync_copy(out_v, out_hbm.at[pl.ds(base, B), :])

    @pl.when(wid == NW - 1)
    def _():
        tail = N - (NW - 1) * B
        pltpu.sync_copy(out_v.at[pl.ds(0, tail), :],
                        out_hbm.at[pl.ds(base, tail), :])


def kernel(h_i, context_indices, W_i, W_j, att_w, att_b):
    idx = context_indices.astype(jnp.int32)
    h_pad = jnp.concatenate(
        [h_i, jnp.zeros((NP - N, F), h_i.dtype)], axis=0)
    g, t, s = _tc_call(h_pad, W_i, W_j, att_w, att_b)
    idx_pad = jnp.concatenate(
        [idx, jnp.zeros((NP - N, C), jnp.int32)], axis=0)
    idx_blocks = idx_pad.reshape(NW, IDX_ROWS, 128)
    g3 = g.reshape(NP // 128, 128, D)
    return _sc_kernel(g, g3, idx_blocks, t, s.reshape(NW, HC, 16))


# speedup vs baseline: 1.1694x; 1.1694x over previous
import functools

import jax
import jax.numpy as jnp
from jax import lax
from jax.experimental import pallas as pl
from jax.experimental.pallas import tpu as pltpu
from jax.experimental.pallas import tpu_sc as plsc

N = 10000
C = 32
F = 128
D = 32
ALPHA = 0.2

NW = 32
B = 320
NP = NW * B
IDX_ROWS = (B * C) // 128
HC = B // 16
OC = B // 8
GPH = 2


def _tc_body(h_ref, wi_ref, wj_ref, aw_ref, ab_ref, g_ref, t_ref, s_ref):
    h = h_ref[...]
    g = lax.dot_general(h, wj_ref[...], (((1,), (1,)), ((), ())),
                        preferred_element_type=jnp.float32)
    g_ref[...] = g
    a_i = aw_ref[0, 0:D]
    a_j = aw_ref[0, D:2 * D]
    t_ref[...] = jnp.sum(g * a_j[None, :], axis=1)
    wh = lax.dot_general(h, wi_ref[...], (((1,), (1,)), ((), ())),
                         preferred_element_type=jnp.float32)
    s_ref[...] = jnp.sum(wh * a_i[None, :], axis=1) + ab_ref[0, 0]


def _tc_call(h_pad, W_i, W_j, att_w, att_b):
    BM = 1024
    return pl.pallas_call(
        _tc_body,
        grid=(NP // BM,),
        in_specs=[
            pl.BlockSpec((BM, F), lambda i: (i, 0)),
            pl.BlockSpec((D, F), lambda i: (0, 0)),
            pl.BlockSpec((D, F), lambda i: (0, 0)),
            pl.BlockSpec((1, 2 * D), lambda i: (0, 0)),
            pl.BlockSpec((1, 1), lambda i: (0, 0)),
        ],
        out_specs=[
            pl.BlockSpec((BM, D), lambda i: (i, 0)),
            pl.BlockSpec((BM,), lambda i: (i,)),
            pl.BlockSpec((BM,), lambda i: (i,)),
        ],
        out_shape=[
            jax.ShapeDtypeStruct((NP, D), jnp.float32),
            jax.ShapeDtypeStruct((NP,), jnp.float32),
            jax.ShapeDtypeStruct((NP,), jnp.float32),
        ],
    )(h_pad, W_i, W_j, att_w, att_b.reshape(1, 1))


@functools.partial(
    pl.kernel,
    mesh=plsc.VectorSubcoreMesh(core_axis_name="c", subcore_axis_name="s"),
    compiler_params=pltpu.CompilerParams(
        needs_layout_passes=False, use_tc_tiling_on_sc=False),
    out_type=jax.ShapeDtypeStruct((NW, B, D), jnp.float32),
    scratch_types=[
        pltpu.VMEM((IDX_ROWS, 128), jnp.int32),
        pltpu.VMEM((HC, 16), jnp.float32),
        pltpu.VMEM((NP,), jnp.float32),
        pltpu.VMEM((GPH, 128, D), jnp.float32),
        pltpu.VMEM((GPH, 128, D), jnp.float32),
        pltpu.VMEM((B, D), jnp.float32),
        pltpu.SemaphoreType.DMA,
        pltpu.SemaphoreType.DMA,
    ],
)
def _sc_kernel(g_hbm, g3_hbm, idx_hbm, t_hbm, s_hbm, out_hbm,
               idx_v, s_v, t_v, rows_a, rows_b, out_v, sem_a, sem_b):
    wid = lax.axis_index("s") * 2 + lax.axis_index("c")
    pltpu.sync_copy(idx_hbm.at[wid], idx_v)
    pltpu.sync_copy(s_hbm.at[wid], s_v)
    pltpu.sync_copy(t_hbm, t_v)

    def fire(oc, rows_v, sem):
        occ = jnp.minimum(oc, OC - 1)
        for j in range(GPH):
            pltpu.async_copy(
                g_hbm.at[idx_v.at[occ * GPH + j]], rows_v.at[j], sem)

    def drain(rows_v, sem):
        for j in range(GPH):
            pltpu.make_async_copy(g3_hbm.at[0], rows_v.at[j], sem).wait()

    def compute(q, half, rows_v):
        sv = s_v[q, :]
        for l in range(8):
            g = q * 16 + half * 8 + l
            ir, ic = q * 4 + half * 2 + l // 4, (l % 4) * 32
            i0 = idx_v[ir, pl.ds(ic, 16)]
            i1 = idx_v[ir, pl.ds(ic + 16, 16)]
            tv0 = plsc.load_gather(t_v, [i0])
            tv1 = plsc.load_gather(t_v, [i1])
            sg = sv[half * 8 + l]
            l0 = sg + tv0
            l1 = sg + tv1
            l0 = jnp.where(l0 >= 0, l0, ALPHA * l0)
            l1 = jnp.where(l1 >= 0, l1, ALPHA * l1)
            e0 = jnp.exp(l0)
            e1 = jnp.exp(l1)
            z = jnp.sum(e0 + e1)
            rinv = jnp.full((16,), 1.0, jnp.float32) / z
            a0 = [jnp.zeros((16,), jnp.float32) for _ in range(4)]
            a1 = [jnp.zeros((16,), jnp.float32) for _ in range(4)]
            for c in range(C):
                wc = e0[c] if c < 16 else e1[c - 16]
                rr, rc = (l * C + c) // 128, (l * C + c) % 128
                a0[c % 4] = a0[c % 4] + wc * rows_v[rr, rc, pl.ds(0, 16)]
                a1[c % 4] = a1[c % 4] + wc * rows_v[rr, rc, pl.ds(16, 16)]
            acc0 = (a0[0] + a0[1]) + (a0[2] + a0[3])
            acc1 = (a1[0] + a1[1]) + (a1[2] + a1[3])
            out_v[g, pl.ds(0, 16)] = acc0 * rinv
            out_v[g, pl.ds(16, 16)] = acc1 * rinv

    fire(0, rows_a, sem_a)

    def body(q, carry):
        oc = 2 * q
        fire(oc + 1, rows_b, sem_b)
        drain(rows_a, sem_a)
        compute(q, 0, rows_a)
        fire(oc + 2, rows_a, sem_a)
        drain(rows_b, sem_b)
        compute(q, 1, rows_b)
        return carry

    lax.fori_loop(0, OC // 2, body, 0)
    drain(rows_a, sem_a)
    pltpu.sync_copy(out_v, out_hbm.at[wid])


def kernel(h_i, context_indices, W_i, W_j, att_w, att_b):
    idx = context_indices.astype(jnp.int32)
    g, t, s = _tc_call(h_i, W_i, W_j, att_w, att_b)
    idx_pad = jnp.concatenate(
        [idx, jnp.zeros((NP - N, C), jnp.int32)], axis=0)
    idx_blocks = idx_pad.reshape(NW, IDX_ROWS, 128)
    g3 = g.reshape(NP // 128, 128, D)
    out = _sc_kernel(g, g3, idx_blocks, t, s.reshape(NW, HC, 16))
    return out.reshape(NP, D)[:N]


# speedup vs baseline: 1.2106x; 1.0352x over previous
import functools

import jax
import jax.numpy as jnp
from jax import lax
from jax.experimental import pallas as pl
from jax.experimental.pallas import tpu as pltpu
from jax.experimental.pallas import tpu_sc as plsc

N = 10000
C = 32
F = 128
D = 32
ALPHA = 0.2

NW = 32
B = 320
NP = NW * B
IDX_ROWS = (B * C) // 128
HC = B // 16
OC = B // 8
GPH = 2


def _tc_body(h_ref, wi_ref, wj_ref, aw_ref, ab_ref, g_ref, t_ref, s_ref):
    h = h_ref[...]
    g = lax.dot_general(h, wj_ref[...], (((1,), (1,)), ((), ())),
                        preferred_element_type=jnp.float32)
    g_ref[...] = g
    a_i = aw_ref[0, 0:D]
    a_j = aw_ref[0, D:2 * D]
    t_ref[...] = jnp.sum(g * a_j[None, :], axis=1)
    wh = lax.dot_general(h, wi_ref[...], (((1,), (1,)), ((), ())),
                         preferred_element_type=jnp.float32)
    s_ref[...] = jnp.sum(wh * a_i[None, :], axis=1) + ab_ref[0, 0]


def _tc_call(h_pad, W_i, W_j, att_w, att_b):
    BM = 1024
    return pl.pallas_call(
        _tc_body,
        grid=(NP // BM,),
        in_specs=[
            pl.BlockSpec((BM, F), lambda i: (i, 0)),
            pl.BlockSpec((D, F), lambda i: (0, 0)),
            pl.BlockSpec((D, F), lambda i: (0, 0)),
            pl.BlockSpec((1, 2 * D), lambda i: (0, 0)),
            pl.BlockSpec((1, 1), lambda i: (0, 0)),
        ],
        out_specs=[
            pl.BlockSpec((BM, D), lambda i: (i, 0)),
            pl.BlockSpec((BM,), lambda i: (i,)),
            pl.BlockSpec((BM,), lambda i: (i,)),
        ],
        out_shape=[
            jax.ShapeDtypeStruct((NP, D), jnp.float32),
            jax.ShapeDtypeStruct((NP,), jnp.float32),
            jax.ShapeDtypeStruct((NP,), jnp.float32),
        ],
    )(h_pad, W_i, W_j, att_w, att_b.reshape(1, 1))


@functools.partial(
    pl.kernel,
    mesh=plsc.VectorSubcoreMesh(core_axis_name="c", subcore_axis_name="s"),
    compiler_params=pltpu.CompilerParams(
        needs_layout_passes=False, use_tc_tiling_on_sc=False),
    out_type=jax.ShapeDtypeStruct((NW, B, D), jnp.float32),
    scratch_types=[
        pltpu.VMEM((IDX_ROWS, 128), jnp.int32),
        pltpu.VMEM((HC, 16), jnp.float32),
        pltpu.VMEM((NP,), jnp.float32),
        pltpu.VMEM((GPH, 128, D), jnp.float32),
        pltpu.VMEM((GPH, 128, D), jnp.float32),
        pltpu.VMEM((B, D), jnp.float32),
        pltpu.SemaphoreType.DMA,
        pltpu.SemaphoreType.DMA,
    ],
)
def _sc_kernel(g_hbm, g3_hbm, idx_hbm, t_hbm, s_hbm, out_hbm,
               idx_v, s_v, t_v, rows_a, rows_b, out_v, sem_a, sem_b):
    wid = lax.axis_index("s") * 2 + lax.axis_index("c")
    pltpu.sync_copy(idx_hbm.at[wid], idx_v)

    def fire(oc, rows_v, sem):
        occ = jnp.minimum(oc, OC - 1)
        for j in range(GPH):
            pltpu.async_copy(
                g_hbm.at[idx_v.at[occ * GPH + j]], rows_v.at[j], sem)

    def drain(rows_v, sem):
        for j in range(GPH):
            pltpu.make_async_copy(g3_hbm.at[0], rows_v.at[j], sem).wait()

    def compute(q, half, rows_v):
        sv = s_v[q, :]
        for l in range(8):
            g = q * 16 + half * 8 + l
            ir, ic = q * 4 + half * 2 + l // 4, (l % 4) * 32
            i0 = idx_v[ir, pl.ds(ic, 16)]
            i1 = idx_v[ir, pl.ds(ic + 16, 16)]
            tv0 = plsc.load_gather(t_v, [i0])
            tv1 = plsc.load_gather(t_v, [i1])
            sg = sv[half * 8 + l]
            l0 = sg + tv0
            l1 = sg + tv1
            l0 = jnp.where(l0 >= 0, l0, ALPHA * l0)
            l1 = jnp.where(l1 >= 0, l1, ALPHA * l1)
            e0 = jnp.exp(l0)
            e1 = jnp.exp(l1)
            z = jnp.sum(e0 + e1)
            rinv = jnp.full((16,), 1.0, jnp.float32) / z
            a0 = [jnp.zeros((16,), jnp.float32) for _ in range(4)]
            a1 = [jnp.zeros((16,), jnp.float32) for _ in range(4)]
            for c in range(C):
                wc = e0[c] if c < 16 else e1[c - 16]
                rr, rc = (l * C + c) // 128, (l * C + c) % 128
                a0[c % 4] = a0[c % 4] + wc * rows_v[rr, rc, pl.ds(0, 16)]
                a1[c % 4] = a1[c % 4] + wc * rows_v[rr, rc, pl.ds(16, 16)]
            acc0 = (a0[0] + a0[1]) + (a0[2] + a0[3])
            acc1 = (a1[0] + a1[1]) + (a1[2] + a1[3])
            out_v[g, pl.ds(0, 16)] = acc0 * rinv
            out_v[g, pl.ds(16, 16)] = acc1 * rinv

    fire(0, rows_a, sem_a)
    pltpu.sync_copy(s_hbm.at[wid], s_v)
    pltpu.sync_copy(t_hbm, t_v)

    def body(q, carry):
        oc = 2 * q
        fire(oc + 1, rows_b, sem_b)
        drain(rows_a, sem_a)
        compute(q, 0, rows_a)
        fire(oc + 2, rows_a, sem_a)
        drain(rows_b, sem_b)
        compute(q, 1, rows_b)
        return carry

    lax.fori_loop(0, OC // 2, body, 0)
    drain(rows_a, sem_a)
    pltpu.sync_copy(out_v, out_hbm.at[wid])


def kernel(h_i, context_indices, W_i, W_j, att_w, att_b):
    idx = context_indices.astype(jnp.int32)
    h_pad = jnp.concatenate(
        [h_i, jnp.zeros((NP - N, F), h_i.dtype)], axis=0)
    g, t, s = _tc_call(h_pad, W_i, W_j, att_w, att_b)
    idx_pad = jnp.concatenate(
        [idx, jnp.zeros((NP - N, C), jnp.int32)], axis=0)
    idx_blocks = idx_pad.reshape(NW, IDX_ROWS, 128)
    g3 = g.reshape(NP // 128, 128, D)
    out = _sc_kernel(g, g3, idx_blocks, t, s.reshape(NW, HC, 16))
    return out.reshape(NP, D)[:N]


# speedup vs baseline: 2.1973x; 1.8151x over previous
import functools

import jax
import jax.numpy as jnp
from jax import lax
from jax.experimental import pallas as pl
from jax.experimental.pallas import tpu as pltpu
from jax.experimental.pallas import tpu_sc as plsc

N = 10000
C = 32
F = 128
D = 32
ALPHA = 0.2

NW = 32
B = 320
NP = NW * B
IDX_ROWS = (B * C) // 128
HC = B // 16
OC = B // 8
GPH = 2


def _tc_body(h_ref, wi_ref, wj_ref, aw_ref, ab_ref, g_ref, t_ref, s_ref):
    h = h_ref[...]
    g = lax.dot_general(h, wj_ref[...], (((1,), (1,)), ((), ())),
                        preferred_element_type=jnp.float32)
    g_ref[...] = g
    a_i = aw_ref[0, 0:D]
    a_j = aw_ref[0, D:2 * D]
    t_ref[...] = jnp.sum(g * a_j[None, :], axis=1)
    wh = lax.dot_general(h, wi_ref[...], (((1,), (1,)), ((), ())),
                         preferred_element_type=jnp.float32)
    s_ref[...] = jnp.sum(wh * a_i[None, :], axis=1) + ab_ref[0, 0]


def _tc_call(h_pad, W_i, W_j, att_w, att_b):
    BM = 1024
    return pl.pallas_call(
        _tc_body,
        grid=(NP // BM,),
        in_specs=[
            pl.BlockSpec((BM, F), lambda i: (i, 0)),
            pl.BlockSpec((D, F), lambda i: (0, 0)),
            pl.BlockSpec((D, F), lambda i: (0, 0)),
            pl.BlockSpec((1, 2 * D), lambda i: (0, 0)),
            pl.BlockSpec((1, 1), lambda i: (0, 0)),
        ],
        out_specs=[
            pl.BlockSpec((BM, D), lambda i: (i, 0)),
            pl.BlockSpec((BM,), lambda i: (i,)),
            pl.BlockSpec((BM,), lambda i: (i,)),
        ],
        out_shape=[
            jax.ShapeDtypeStruct((NP, D), jnp.float32),
            jax.ShapeDtypeStruct((NP,), jnp.float32),
            jax.ShapeDtypeStruct((NP,), jnp.float32),
        ],
    )(h_pad, W_i, W_j, att_w, att_b.reshape(1, 1))


@functools.partial(
    pl.kernel,
    mesh=plsc.VectorSubcoreMesh(core_axis_name="c", subcore_axis_name="s"),
    compiler_params=pltpu.CompilerParams(
        needs_layout_passes=False, use_tc_tiling_on_sc=False),
    out_type=jax.ShapeDtypeStruct((NW, B, D), jnp.float32),
    scratch_types=[
        pltpu.VMEM((IDX_ROWS, 128), jnp.int32),
        pltpu.VMEM((HC, 16), jnp.float32),
        pltpu.VMEM((NP,), jnp.float32),
        pltpu.VMEM((GPH, 128, D), jnp.float32),
        pltpu.VMEM((GPH, 128, D), jnp.float32),
        pltpu.VMEM((B, D), jnp.float32),
        pltpu.VMEM_SHARED((NP, D), jnp.float32),
        pltpu.SemaphoreType.DMA,
        pltpu.SemaphoreType.DMA,
    ],
)
def _sc_kernel(g_hbm, g3_hbm, idx_hbm, t_hbm, s_hbm, out_hbm,
               idx_v, s_v, t_v, rows_a, rows_b, out_v, g_sh, sem_a, sem_b):
    wid = lax.axis_index("s") * 2 + lax.axis_index("c")

    @pl.when(lax.axis_index("s") == 0)
    def _():
        pltpu.sync_copy(g_hbm, g_sh)

    pltpu.sync_copy(idx_hbm.at[wid], idx_v)
    plsc.subcore_barrier()

    def fire(oc, rows_v, sem):
        occ = jnp.minimum(oc, OC - 1)
        for j in range(GPH):
            pltpu.async_copy(
                g_sh.at[idx_v.at[occ * GPH + j]], rows_v.at[j], sem)

    def drain(rows_v, sem):
        for j in range(GPH):
            pltpu.make_async_copy(g3_hbm.at[0], rows_v.at[j], sem).wait()

    def compute(q, half, rows_v):
        sv = s_v[q, :]
        for l in range(8):
            g = q * 16 + half * 8 + l
            ir, ic = q * 4 + half * 2 + l // 4, (l % 4) * 32
            i0 = idx_v[ir, pl.ds(ic, 16)]
            i1 = idx_v[ir, pl.ds(ic + 16, 16)]
            tv0 = plsc.load_gather(t_v, [i0])
            tv1 = plsc.load_gather(t_v, [i1])
            sg = sv[half * 8 + l]
            l0 = sg + tv0
            l1 = sg + tv1
            l0 = jnp.where(l0 >= 0, l0, ALPHA * l0)
            l1 = jnp.where(l1 >= 0, l1, ALPHA * l1)
            e0 = jnp.exp(l0)
            e1 = jnp.exp(l1)
            z = jnp.sum(e0 + e1)
            rinv = jnp.full((16,), 1.0, jnp.float32) / z
            a0 = [jnp.zeros((16,), jnp.float32) for _ in range(4)]
            a1 = [jnp.zeros((16,), jnp.float32) for _ in range(4)]
            for c in range(C):
                wc = e0[c] if c < 16 else e1[c - 16]
                rr, rc = (l * C + c) // 128, (l * C + c) % 128
                a0[c % 4] = a0[c % 4] + wc * rows_v[rr, rc, pl.ds(0, 16)]
                a1[c % 4] = a1[c % 4] + wc * rows_v[rr, rc, pl.ds(16, 16)]
            acc0 = (a0[0] + a0[1]) + (a0[2] + a0[3])
            acc1 = (a1[0] + a1[1]) + (a1[2] + a1[3])
            out_v[g, pl.ds(0, 16)] = acc0 * rinv
            out_v[g, pl.ds(16, 16)] = acc1 * rinv

    fire(0, rows_a, sem_a)
    pltpu.sync_copy(s_hbm.at[wid], s_v)
    pltpu.sync_copy(t_hbm, t_v)

    def body(q, carry):
        oc = 2 * q
        fire(oc + 1, rows_b, sem_b)
        drain(rows_a, sem_a)
        compute(q, 0, rows_a)
        fire(oc + 2, rows_a, sem_a)
        drain(rows_b, sem_b)
        compute(q, 1, rows_b)
        return carry

    lax.fori_loop(0, OC // 2, body, 0)
    drain(rows_a, sem_a)
    pltpu.sync_copy(out_v, out_hbm.at[wid])


def kernel(h_i, context_indices, W_i, W_j, att_w, att_b):
    idx = context_indices.astype(jnp.int32)
    h_pad = jnp.concatenate(
        [h_i, jnp.zeros((NP - N, F), h_i.dtype)], axis=0)
    g, t, s = _tc_call(h_pad, W_i, W_j, att_w, att_b)
    idx_pad = jnp.concatenate(
        [idx, jnp.zeros((NP - N, C), jnp.int32)], axis=0)
    idx_blocks = idx_pad.reshape(NW, IDX_ROWS, 128)
    g3 = g.reshape(NP // 128, 128, D)
    out = _sc_kernel(g, g3, idx_blocks, t, s.reshape(NW, HC, 16))
    return out.reshape(NP, D)[:N]
